# baseline (device time: 241648 ns/iter reference)
import jax
import jax.numpy as jnp
from jax import lax
from jax.experimental import pallas as pl
from jax.experimental.pallas import tpu as pltpu

N_BLOCKS = 16
BLK_N = 8192 // N_BLOCKS
QROWS = 256
K_SHARD = 4096
NSLOT = 8
MESH = pl.DeviceIdType.MESH


def kernel(O, Wo):
    O2 = O.reshape(2048, K_SHARD).astype(jnp.bfloat16)
    xi = lax.axis_index("x")
    zi = lax.axis_index("z")
    cq = 2 * xi + zi
    Oq = jnp.concatenate([
        lax.dynamic_slice(O2, (QROWS * cq, 0), (QROWS, K_SHARD)),
        lax.dynamic_slice(O2, (1024 + QROWS * cq, 0), (QROWS, K_SHARD)),
    ], axis=0)

    def body(o_ref, wo_ref, out_ref,
             pbuf, yrecv, qbuf, q1recv, q2recv_a, q2recv_b,
             ysend_sems, yrecv_sems, fsend_sems, frecv_sems,
             sasend_sems, sarecv_sems, sbsend_sems, sbrecv_sems,
             credit_y, credit_1e, credit_1o, credit_2e, credit_2o):
        s = pl.program_id(0)
        x = lax.axis_index("x")
        y = lax.axis_index("y")
        z = lax.axis_index("z")
        c = 2 * x + z
        y_partner = (x, 1 - y, z)

        def plane_partners(b):
            p = lax.rem(b, 2)
            first = (jnp.where(p == 0, 1 - x, x), y,
                     jnp.where(p == 0, z, 1 - z))
            second = (jnp.where(p == 0, x, 1 - x), y,
                      jnp.where(p == 0, 1 - z, z))
            return first, second

        def rdma_y(sl):
            return pltpu.make_async_remote_copy(
                src_ref=pbuf.at[sl, pl.ds(QROWS * (1 - y), QROWS)],
                dst_ref=yrecv.at[sl],
                send_sem=ysend_sems.at[sl], recv_sem=yrecv_sems.at[sl],
                device_id=y_partner, device_id_type=MESH)

        def rdma_first(sl, b):
            first, _ = plane_partners(b)
            return pltpu.make_async_remote_copy(
                src_ref=qbuf.at[sl], dst_ref=q1recv.at[sl],
                send_sem=fsend_sems.at[sl], recv_sem=frecv_sems.at[sl],
                device_id=first, device_id_type=MESH)

        def rdma_2a(sl, b):
            _, second = plane_partners(b)
            return pltpu.make_async_remote_copy(
                src_ref=qbuf.at[sl], dst_ref=q2recv_a.at[sl],
                send_sem=sasend_sems.at[sl], recv_sem=sarecv_sems.at[sl],
                device_id=second, device_id_type=MESH)

        def rdma_2b(sl, b):
            _, second = plane_partners(b)
            return pltpu.make_async_remote_copy(
                src_ref=q1recv.at[sl], dst_ref=q2recv_b.at[sl],
                send_sem=sbsend_sems.at[sl], recv_sem=sbrecv_sems.at[sl],
                device_id=second, device_id_type=MESH)

        @pl.when(s == 0)
        def _():
            bar = pltpu.get_barrier_semaphore()
            for nbr in (y_partner, (1 - x, y, z), (x, y, 1 - z)):
                pl.semaphore_signal(bar, inc=1, device_id=nbr,
                                    device_id_type=MESH)
            pl.semaphore_wait(bar, 3)

        @pl.when(s < N_BLOCKS)
        def _():
            sl = lax.rem(s, NSLOT)

            @pl.when(s >= NSLOT)
            def _():
                rdma_y(sl).wait_send()

            pbuf[sl] = jnp.dot(
                o_ref[...], wo_ref[...],
                preferred_element_type=jnp.float32)

            @pl.when(s >= NSLOT)
            def _():
                pl.semaphore_wait(credit_y, 1)

            rdma_y(sl).start()

        @pl.when(jnp.logical_and(s >= 1, s <= N_BLOCKS))
        def _():
            b = s - 1
            sl = lax.rem(b, NSLOT)
            rdma_y(sl).wait_recv()

            @pl.when(b >= NSLOT)
            def _():
                rdma_first(sl, b).wait_send()

            qbuf[sl] = pbuf[sl, pl.ds(QROWS * y, QROWS), :] + yrecv[sl]
            pl.semaphore_signal(credit_y, inc=1, device_id=y_partner,
                                device_id_type=MESH)

            @pl.when(jnp.logical_and(b >= NSLOT, lax.rem(b, 2) == 0))
            def _():
                pl.semaphore_wait(credit_1e, 1)

            @pl.when(jnp.logical_and(b >= NSLOT, lax.rem(b, 2) == 1))
            def _():
                pl.semaphore_wait(credit_1o, 1)

            rdma_first(sl, b).start()

        @pl.when(jnp.logical_and(s >= 2, s <= N_BLOCKS + 1))
        def _():
            b = s - 2
            sl = lax.rem(b, NSLOT)
            rdma_first(sl, b).wait_recv()

            @pl.when(jnp.logical_and(b >= NSLOT, lax.rem(b, 2) == 0))
            def _():
                pl.semaphore_wait(credit_2e, 1)

            @pl.when(jnp.logical_and(b >= NSLOT, lax.rem(b, 2) == 1))
            def _():
                pl.semaphore_wait(credit_2o, 1)

            rdma_2a(sl, b).start()
            rdma_2b(sl, b).start()

        @pl.when(s >= 3)
        def _():
            b = s - 3
            sl = lax.rem(b, NSLOT)
            p = lax.rem(b, 2)
            m1 = jnp.where(p == 0, 2, 1)
            rdma_2a(sl, b).wait_recv()
            rdma_2b(sl, b).wait_recv()
            rdma_2a(sl, b).wait_send()
            rdma_2b(sl, b).wait_send()

            out_ref[pl.ds(QROWS * c, QROWS), :] = qbuf[sl]
            out_ref[pl.ds(QROWS * jnp.bitwise_xor(c, m1), QROWS), :] = (
                q1recv[sl])
            out_ref[pl.ds(QROWS * jnp.bitwise_xor(c, 3 - m1), QROWS), :] = (
                q2recv_a[sl])
            out_ref[pl.ds(QROWS * jnp.bitwise_xor(c, 3), QROWS), :] = (
                q2recv_b[sl])

            first, second = plane_partners(b)

            @pl.when(p == 0)
            def _():
                pl.semaphore_signal(credit_1e, inc=1, device_id=first,
                                    device_id_type=MESH)
                pl.semaphore_signal(credit_2e, inc=1, device_id=second,
                                    device_id_type=MESH)

            @pl.when(p == 1)
            def _():
                pl.semaphore_signal(credit_1o, inc=1, device_id=first,
                                    device_id_type=MESH)
                pl.semaphore_signal(credit_2o, inc=1, device_id=second,
                                    device_id_type=MESH)

        @pl.when(s == N_BLOCKS + 2)
        def _():
            pl.semaphore_wait(credit_y, NSLOT)
            pl.semaphore_wait(credit_1e, NSLOT // 2)
            pl.semaphore_wait(credit_1o, NSLOT // 2)
            pl.semaphore_wait(credit_2e, NSLOT // 2)
            pl.semaphore_wait(credit_2o, NSLOT // 2)
            for i in range(NSLOT):
                rdma_y(i).wait_send()
                rdma_first(i, i).wait_send()

    last = N_BLOCKS - 1
    out = pl.pallas_call(
        body,
        grid=(N_BLOCKS + 3,),
        in_specs=[
            pl.BlockSpec((2 * QROWS, K_SHARD), lambda s: (0, 0)),
            pl.BlockSpec((K_SHARD, BLK_N), lambda s: (0, jnp.minimum(s, last))),
        ],
        out_specs=pl.BlockSpec(
            (4 * QROWS, BLK_N), lambda s: (0, jnp.maximum(s - 3, 0))
        ),
        out_shape=jax.ShapeDtypeStruct((4 * QROWS, 8192), jnp.float32),
        scratch_shapes=[
            pltpu.VMEM((NSLOT, 2 * QROWS, BLK_N), jnp.float32),
            pltpu.VMEM((NSLOT, QROWS, BLK_N), jnp.float32),
            pltpu.VMEM((NSLOT, QROWS, BLK_N), jnp.float32),
            pltpu.VMEM((NSLOT, QROWS, BLK_N), jnp.float32),
            pltpu.VMEM((NSLOT, QROWS, BLK_N), jnp.float32),
            pltpu.VMEM((NSLOT, QROWS, BLK_N), jnp.float32),
            pltpu.SemaphoreType.DMA((NSLOT,)),
            pltpu.SemaphoreType.DMA((NSLOT,)),
            pltpu.SemaphoreType.DMA((NSLOT,)),
            pltpu.SemaphoreType.DMA((NSLOT,)),
            pltpu.SemaphoreType.DMA((NSLOT,)),
            pltpu.SemaphoreType.DMA((NSLOT,)),
            pltpu.SemaphoreType.DMA((NSLOT,)),
            pltpu.SemaphoreType.DMA((NSLOT,)),
            pltpu.SemaphoreType.REGULAR,
            pltpu.SemaphoreType.REGULAR,
            pltpu.SemaphoreType.REGULAR,
            pltpu.SemaphoreType.REGULAR,
            pltpu.SemaphoreType.REGULAR,
        ],
        compiler_params=pltpu.CompilerParams(
            collective_id=0,
            dimension_semantics=("arbitrary",),
            vmem_limit_bytes=60 * 1024 * 1024,
        ),
    )(Oq, Wo)
    return out.reshape(1, 4 * QROWS, 8192)


# device time: 240901 ns/iter; 1.0031x vs baseline; 1.0031x over previous
import jax
import jax.numpy as jnp
from jax import lax
from jax.experimental import pallas as pl
from jax.experimental.pallas import tpu as pltpu

N_BLOCKS = 16
BLK_N = 8192 // N_BLOCKS
QROWS = 256
K_SHARD = 4096
NSLOT = 4
MESH = pl.DeviceIdType.MESH


def kernel(O, Wo):
    O2 = O.reshape(2048, K_SHARD).astype(jnp.bfloat16)
    xi = lax.axis_index("x")
    zi = lax.axis_index("z")
    cq = 2 * xi + zi
    Oq = jnp.concatenate([
        lax.dynamic_slice(O2, (QROWS * cq, 0), (QROWS, K_SHARD)),
        lax.dynamic_slice(O2, (1024 + QROWS * cq, 0), (QROWS, K_SHARD)),
    ], axis=0)

    def body(o_ref, wo_ref, out_ref,
             pbuf, yrecv, qbuf, q1recv, q2recv_a, q2recv_b,
             ysend_sems, yrecv_sems, fsend_sems, frecv_sems,
             sasend_sems, sarecv_sems, sbsend_sems, sbrecv_sems,
             credit_y, credit_1e, credit_1o, credit_2e, credit_2o):
        s = pl.program_id(0)
        x = lax.axis_index("x")
        y = lax.axis_index("y")
        z = lax.axis_index("z")
        c = 2 * x + z
        y_partner = (x, 1 - y, z)

        def plane_partners(b):
            p = lax.rem(b, 2)
            first = (jnp.where(p == 0, 1 - x, x), y,
                     jnp.where(p == 0, z, 1 - z))
            second = (jnp.where(p == 0, x, 1 - x), y,
                      jnp.where(p == 0, 1 - z, z))
            return first, second

        def rdma_y(sl):
            return pltpu.make_async_remote_copy(
                src_ref=pbuf.at[sl, pl.ds(QROWS * (1 - y), QROWS)],
                dst_ref=yrecv.at[sl],
                send_sem=ysend_sems.at[sl], recv_sem=yrecv_sems.at[sl],
                device_id=y_partner, device_id_type=MESH)

        def rdma_first(sl, b):
            first, _ = plane_partners(b)
            return pltpu.make_async_remote_copy(
                src_ref=qbuf.at[sl], dst_ref=q1recv.at[sl],
                send_sem=fsend_sems.at[sl], recv_sem=frecv_sems.at[sl],
                device_id=first, device_id_type=MESH)

        def rdma_2a(sl, b):
            _, second = plane_partners(b)
            return pltpu.make_async_remote_copy(
                src_ref=qbuf.at[sl], dst_ref=q2recv_a.at[sl],
                send_sem=sasend_sems.at[sl], recv_sem=sarecv_sems.at[sl],
                device_id=second, device_id_type=MESH)

        def rdma_2b(sl, b):
            _, second = plane_partners(b)
            return pltpu.make_async_remote_copy(
                src_ref=q1recv.at[sl], dst_ref=q2recv_b.at[sl],
                send_sem=sbsend_sems.at[sl], recv_sem=sbrecv_sems.at[sl],
                device_id=second, device_id_type=MESH)

        @pl.when(s == 0)
        def _():
            bar = pltpu.get_barrier_semaphore()
            for nbr in (y_partner, (1 - x, y, z), (x, y, 1 - z)):
                pl.semaphore_signal(bar, inc=1, device_id=nbr,
                                    device_id_type=MESH)
            pl.semaphore_wait(bar, 3)

        @pl.when(s < N_BLOCKS)
        def _():
            sl = lax.rem(s, NSLOT)

            @pl.when(s >= NSLOT)
            def _():
                rdma_y(sl).wait_send()

            pbuf[sl] = jnp.dot(
                o_ref[...], wo_ref[...],
                preferred_element_type=jnp.float32)

            @pl.when(s >= NSLOT)
            def _():
                pl.semaphore_wait(credit_y, 1)

            rdma_y(sl).start()

        @pl.when(jnp.logical_and(s >= 1, s <= N_BLOCKS))
        def _():
            b = s - 1
            sl = lax.rem(b, NSLOT)
            rdma_y(sl).wait_recv()

            @pl.when(b >= NSLOT)
            def _():
                rdma_first(sl, b).wait_send()

            qbuf[sl] = pbuf[sl, pl.ds(QROWS * y, QROWS), :] + yrecv[sl]
            pl.semaphore_signal(credit_y, inc=1, device_id=y_partner,
                                device_id_type=MESH)

            @pl.when(jnp.logical_and(b >= NSLOT, lax.rem(b, 2) == 0))
            def _():
                pl.semaphore_wait(credit_1e, 1)

            @pl.when(jnp.logical_and(b >= NSLOT, lax.rem(b, 2) == 1))
            def _():
                pl.semaphore_wait(credit_1o, 1)

            rdma_first(sl, b).start()

        @pl.when(jnp.logical_and(s >= 2, s <= N_BLOCKS + 1))
        def _():
            b = s - 2
            sl = lax.rem(b, NSLOT)
            rdma_first(sl, b).wait_recv()

            @pl.when(jnp.logical_and(b >= NSLOT, lax.rem(b, 2) == 0))
            def _():
                pl.semaphore_wait(credit_2e, 1)

            @pl.when(jnp.logical_and(b >= NSLOT, lax.rem(b, 2) == 1))
            def _():
                pl.semaphore_wait(credit_2o, 1)

            rdma_2a(sl, b).start()
            rdma_2b(sl, b).start()

        @pl.when(s >= 3)
        def _():
            b = s - 3
            sl = lax.rem(b, NSLOT)
            p = lax.rem(b, 2)
            m1 = jnp.where(p == 0, 2, 1)
            rdma_2a(sl, b).wait_recv()
            rdma_2b(sl, b).wait_recv()
            rdma_2a(sl, b).wait_send()
            rdma_2b(sl, b).wait_send()

            out_ref[pl.ds(QROWS * c, QROWS), :] = qbuf[sl]
            out_ref[pl.ds(QROWS * jnp.bitwise_xor(c, m1), QROWS), :] = (
                q1recv[sl])
            out_ref[pl.ds(QROWS * jnp.bitwise_xor(c, 3 - m1), QROWS), :] = (
                q2recv_a[sl])
            out_ref[pl.ds(QROWS * jnp.bitwise_xor(c, 3), QROWS), :] = (
                q2recv_b[sl])

            first, second = plane_partners(b)

            @pl.when(p == 0)
            def _():
                pl.semaphore_signal(credit_1e, inc=1, device_id=first,
                                    device_id_type=MESH)
                pl.semaphore_signal(credit_2e, inc=1, device_id=second,
                                    device_id_type=MESH)

            @pl.when(p == 1)
            def _():
                pl.semaphore_signal(credit_1o, inc=1, device_id=first,
                                    device_id_type=MESH)
                pl.semaphore_signal(credit_2o, inc=1, device_id=second,
                                    device_id_type=MESH)

        @pl.when(s == N_BLOCKS + 2)
        def _():
            pl.semaphore_wait(credit_y, NSLOT)
            pl.semaphore_wait(credit_1e, NSLOT // 2)
            pl.semaphore_wait(credit_1o, NSLOT // 2)
            pl.semaphore_wait(credit_2e, NSLOT // 2)
            pl.semaphore_wait(credit_2o, NSLOT // 2)
            for i in range(NSLOT):
                rdma_y(i).wait_send()
                rdma_first(i, i).wait_send()

    last = N_BLOCKS - 1
    out = pl.pallas_call(
        body,
        grid=(N_BLOCKS + 3,),
        in_specs=[
            pl.BlockSpec((2 * QROWS, K_SHARD), lambda s: (0, 0)),
            pl.BlockSpec((K_SHARD, BLK_N), lambda s: (0, jnp.minimum(s, last))),
        ],
        out_specs=pl.BlockSpec(
            (4 * QROWS, BLK_N), lambda s: (0, jnp.maximum(s - 3, 0))
        ),
        out_shape=jax.ShapeDtypeStruct((4 * QROWS, 8192), jnp.float32),
        scratch_shapes=[
            pltpu.VMEM((NSLOT, 2 * QROWS, BLK_N), jnp.float32),
            pltpu.VMEM((NSLOT, QROWS, BLK_N), jnp.float32),
            pltpu.VMEM((NSLOT, QROWS, BLK_N), jnp.float32),
            pltpu.VMEM((NSLOT, QROWS, BLK_N), jnp.float32),
            pltpu.VMEM((NSLOT, QROWS, BLK_N), jnp.float32),
            pltpu.VMEM((NSLOT, QROWS, BLK_N), jnp.float32),
            pltpu.SemaphoreType.DMA((NSLOT,)),
            pltpu.SemaphoreType.DMA((NSLOT,)),
            pltpu.SemaphoreType.DMA((NSLOT,)),
            pltpu.SemaphoreType.DMA((NSLOT,)),
            pltpu.SemaphoreType.DMA((NSLOT,)),
            pltpu.SemaphoreType.DMA((NSLOT,)),
            pltpu.SemaphoreType.DMA((NSLOT,)),
            pltpu.SemaphoreType.DMA((NSLOT,)),
            pltpu.SemaphoreType.REGULAR,
            pltpu.SemaphoreType.REGULAR,
            pltpu.SemaphoreType.REGULAR,
            pltpu.SemaphoreType.REGULAR,
            pltpu.SemaphoreType.REGULAR,
        ],
        compiler_params=pltpu.CompilerParams(
            collective_id=0,
            dimension_semantics=("arbitrary",),
            vmem_limit_bytes=60 * 1024 * 1024,
        ),
    )(Oq, Wo)
    return out.reshape(1, 4 * QROWS, 8192)
